# baseline (speedup 1.0000x reference)
import functools

import jax
import jax.numpy as jnp
from jax import lax
from jax.experimental import pallas as pl
from jax.experimental.pallas import tpu as pltpu
from jax.experimental.pallas import tpu_sc as plsc

N = 10000
E = 320000
D = 128

NC = 2
NS = 16
NW = NC * NS
EPT = E // NW
CH = 40
NCHUNK = EPT // CH
G = 400 // CH
NG = NCHUNK // G
GE = G * CH
R = 5
PF = 2
DCH = 40
DNCHUNK = EPT // DCH
RPS = 624
ZB = 16

_mesh = plsc.VectorSubcoreMesh(core_axis_name="c", subcore_axis_name="s")


def _zero_fill(ref, nrow, ncol):
    zv = jnp.zeros((16,), jnp.float32)

    def body(i, _):
        for c in range(ncol // 16):
            ref[i, pl.ds(c * 16, 16)] = zv
        return 0

    lax.fori_loop(0, nrow, body, 0)


def _zero_acc(zbuf, acc, sid):
    z16 = zbuf.at[pl.ds(0, ZB)]
    for j in range(RPS // ZB):
        pltpu.sync_copy(z16, acc.at[pl.ds(sid * RPS + j * ZB, ZB)])

    @pl.when(sid == NS - 1)
    def _():
        pltpu.sync_copy(z16, acc.at[pl.ds(NS * RPS, 16)])


def _writeback(acc, out_hbm, cid, sid):
    pltpu.sync_copy(
        acc.at[pl.ds(sid * RPS, RPS)],
        out_hbm.at[cid, pl.ds(sid * RPS, RPS)],
    )

    @pl.when(sid == NS - 1)
    def _():
        pltpu.sync_copy(
            acc.at[pl.ds(NS * RPS, 16)],
            out_hbm.at[cid, pl.ds(NS * RPS, 16)],
        )


def _sc_prop_body(y_hbm, src_hbm, dst_hbm, out_hbm, idx_s0, idx_s1, idx_d,
                  rows, acc, sem_i, sem_g, sem_s):
    cid = lax.axis_index("c")
    sid = lax.axis_index("s")
    wid = cid * NS + sid
    idx_s = [idx_s0, idx_s1]

    def fetch_idx(group, p):
        pltpu.async_copy(src_hbm.at[pl.ds(wid * EPT + group * GE, GE)],
                         idx_s[p], sem_i.at[p])
        pltpu.async_copy(dst_hbm.at[wid, group], idx_d.at[p], sem_i.at[p])

    def wait_idx(group, p):
        pltpu.make_async_copy(src_hbm.at[pl.ds(wid * EPT + group * GE, GE)],
                              idx_s[p], sem_i.at[p]).wait()
        pltpu.make_async_copy(dst_hbm.at[wid, group], idx_d.at[p],
                              sem_i.at[p]).wait()

    def start_gather(p, l, b):
        pltpu.async_copy(y_hbm.at[idx_s[p].at[pl.ds(l * CH, CH)]], rows.at[b],
                         sem_g.at[b])

    def wait_gather(p, l, b):
        pltpu.make_async_copy(y_hbm.at[idx_s[p].at[pl.ds(l * CH, CH)]],
                              rows.at[b], sem_g.at[b]).wait()

    def start_scatter(p, l, b):
        pltpu.async_copy(rows.at[b], acc.at[idx_d.at[p, l]], sem_s.at[b],
                         add=True)

    def wait_scatter(p, l, b):
        pltpu.make_async_copy(rows.at[b], acc.at[idx_d.at[p, l]],
                              sem_s.at[b]).wait()

    def slotpos(t):
        return (t // G) % 2, t % G, t % R

    fetch_idx(0, 0)
    zbuf = rows.at[0]
    _zero_fill(zbuf, ZB, D)
    _zero_acc(zbuf, acc, sid)
    wait_idx(0, 0)

    for t in range(PF):
        start_gather(0, t, t)
    plsc.subcore_barrier()

    def step(m, t, first):
        p, l, b = slotpos(t)
        wait_gather(p, l, b)
        start_scatter(p, l, b)
        if t == 2:
            fetch_idx(2 * m + 1, 1)
        if t == G + 2:
            fetch_idx(2 * m + 2, 0)
        tj = t + PF
        if tj == G:
            wait_idx(2 * m + 1, 1)
        if tj == 2 * G:
            wait_idx(2 * m + 2, 0)
        pj, lj, bj = slotpos(tj % (2 * G))
        if not (first and tj - R < 0):
            td = tj - R
            if first:
                pd, ld, bd = slotpos(td)
                wait_scatter(pd, ld, bd)
            else:
                pd, ld, bd = slotpos(td % (2 * G))
                wait_scatter(pd, ld, bd)
        start_gather(pj, lj, bj)

    for t in range(2 * G):
        step(0, t, True)

    def sgroup(m, _):
        for t in range(2 * G):
            step(m, t, False)
        return 0

    nsg = NCHUNK // (2 * G)
    lax.fori_loop(1, nsg, sgroup, 0)

    m = nsg
    for t in range(G):
        p, l, b = slotpos(t)
        wait_gather(p, l, b)
        start_scatter(p, l, b)
        tj = t + PF
        if tj < G:
            td = tj - R
            if td >= 0:
                pd, ld, bd = slotpos(td)
                wait_scatter(pd, ld, bd)
            else:
                pd, ld, bd = slotpos((td) % (2 * G))
                wait_scatter(pd, ld, bd)
            start_gather(0, tj, tj % R)

    for t in range(G - R, G):
        p, l, b = slotpos(t)
        wait_scatter(p, l, b)

    plsc.subcore_barrier()
    _writeback(acc, out_hbm, cid, sid)


_sc_prop = pl.kernel(
    _sc_prop_body,
    out_type=jax.ShapeDtypeStruct((NC, N, D), jnp.float32),
    mesh=_mesh,
    scratch_types=[
        pltpu.VMEM((GE,), jnp.int32),
        pltpu.VMEM((GE,), jnp.int32),
        pltpu.VMEM((2, G, CH), jnp.int32),
        pltpu.VMEM((R, CH, D), jnp.float32),
        pltpu.VMEM_SHARED((N, D), jnp.float32),
        pltpu.SemaphoreType.DMA((2,)),
        pltpu.SemaphoreType.DMA((R,)),
        pltpu.SemaphoreType.DMA((R,)),
    ],
)


def _sc_deg_body(src_hbm, out_hbm, idx_s, ones, zbuf, acc, sem_i, sem_s):
    cid = lax.axis_index("c")
    sid = lax.axis_index("s")
    wid = cid * NS + sid

    ci = pltpu.async_copy(src_hbm.at[wid], idx_s, sem_i)
    _zero_fill(zbuf, ZB, D)
    ov = jnp.ones((16,), jnp.float32)

    def fill_ones(i, _):
        for c in range(D // 16):
            ones[i, pl.ds(c * 16, 16)] = ov
        return 0

    lax.fori_loop(0, DCH, fill_ones, 0)
    _zero_acc(zbuf, acc, sid)
    ci.wait()
    plsc.subcore_barrier()

    def fire(i, _):
        pltpu.async_copy(ones, acc.at[idx_s.at[i]], sem_s, add=True)
        return 0

    def drain(i, _):
        pltpu.make_async_copy(ones, acc.at[idx_s.at[i]], sem_s).wait()
        return 0

    lax.fori_loop(0, DNCHUNK, fire, 0)
    lax.fori_loop(0, DNCHUNK, drain, 0)
    plsc.subcore_barrier()
    _writeback(acc, out_hbm, cid, sid)


_sc_deg = pl.kernel(
    _sc_deg_body,
    out_type=jax.ShapeDtypeStruct((NC, N, D), jnp.float32),
    mesh=_mesh,
    scratch_types=[
        pltpu.VMEM((DNCHUNK, DCH), jnp.int32),
        pltpu.VMEM((DCH, D), jnp.float32),
        pltpu.VMEM((ZB, D), jnp.float32),
        pltpu.VMEM_SHARED((N, D), jnp.float32),
        pltpu.SemaphoreType.DMA,
        pltpu.SemaphoreType.DMA,
    ],
)


def _dis_of(d_ref):
    deg = d_ref[0][:, 0:1] + d_ref[1][:, 0:1]
    return jnp.where(deg > 0.0, lax.rsqrt(jnp.maximum(deg, 1.0)), 0.0)


def _tc_scale_body(d_ref, x_ref, o_ref):
    o_ref[...] = _dis_of(d_ref) * x_ref[...]


_tc_scale = pl.pallas_call(
    _tc_scale_body,
    out_shape=jax.ShapeDtypeStruct((N, D), jnp.float32),
)


def _tc_mid_body(d_ref, p_ref, tx_ref, y_ref):
    dis = _dis_of(d_ref)
    tx1 = (-dis) * (p_ref[0] + p_ref[1])
    tx_ref[...] = tx1
    y_ref[...] = dis * tx1


_tc_mid = pl.pallas_call(
    _tc_mid_body,
    out_shape=(
        jax.ShapeDtypeStruct((N, D), jnp.float32),
        jax.ShapeDtypeStruct((N, D), jnp.float32),
    ),
)


def _tc_layer_body(d_ref, xin_ref, tx1_ref, p_ref, w_ref, b_ref, g_ref, be_ref,
                   *refs, relu):
    dis = _dis_of(d_ref)
    sp = (-dis) * (p_ref[0] + p_ref[1])
    h = (
        jnp.dot(xin_ref[...], w_ref[0] - w_ref[2],
                preferred_element_type=jnp.float32)
        + jnp.dot(tx1_ref[...], w_ref[1], preferred_element_type=jnp.float32)
        + 2.0 * jnp.dot(sp, w_ref[2], preferred_element_type=jnp.float32)
        + b_ref[...]
    )
    m = jnp.mean(h, axis=0, keepdims=True)
    c = h - m
    v = jnp.mean(c * c, axis=0, keepdims=True)
    o = g_ref[...] * c * lax.rsqrt(v + 1e-5) + be_ref[...]
    if relu:
        o = jnp.maximum(o, 0.0)
        refs[0][...] = o
        refs[1][...] = dis * o
    else:
        refs[0][...] = o


_tc_layer_relu = pl.pallas_call(
    functools.partial(_tc_layer_body, relu=True),
    out_shape=(
        jax.ShapeDtypeStruct((N, D), jnp.float32),
        jax.ShapeDtypeStruct((N, D), jnp.float32),
    ),
)
_tc_layer_lin = pl.pallas_call(
    functools.partial(_tc_layer_body, relu=False),
    out_shape=jax.ShapeDtypeStruct((N, D), jnp.float32),
)


def _jnp_ref(x, edge_index, W1, b1, gamma1, beta1, W2, b2, gamma2, beta2,
             deg=None, prop=None):
    row, col = edge_index[0], edge_index[1]
    n = x.shape[0]
    if deg is None:
        deg = jnp.zeros((n,), jnp.float32).at[row].add(1.0)
    dis = jnp.where(deg > 0, lax.rsqrt(jnp.maximum(deg, 1e-12)), 0.0)
    if prop is None:
        def prop(y):
            return jnp.zeros((n, y.shape[1]), jnp.float32).at[col].add(
                jnp.take(y, row, axis=0))

    def S(y):
        return -dis[:, None] * prop(dis[:, None] * y)

    def layer(xin, W, b):
        tx1 = S(xin)
        sp = S(tx1)
        return xin @ (W[0] - W[2]) + tx1 @ W[1] + 2.0 * (sp @ W[2]) + b

    def bn(h, g, b):
        m = jnp.mean(h, axis=0)
        v = jnp.mean((h - m) ** 2, axis=0)
        return g * (h - m) / jnp.sqrt(v + 1e-5) + b

    h = jax.nn.relu(bn(layer(x, W1, b1), gamma1, beta1))
    return bn(layer(h, W2, b2), gamma2, beta2)


def kernel(x, edge_index, W1, b1, gamma1, beta1, W2, b2, gamma2, beta2):
    row3 = edge_index[0].reshape(NW, DNCHUNK, DCH)
    row = edge_index[0]
    col = edge_index[1].reshape(NW, NG, G, CH)
    b1r = b1.reshape(1, D)
    g1r = gamma1.reshape(1, D)
    be1r = beta1.reshape(1, D)
    b2r = b2.reshape(1, D)
    g2r = gamma2.reshape(1, D)
    be2r = beta2.reshape(1, D)

    d2 = _sc_deg(row3)

    y1 = _tc_scale(d2, x)
    p1 = _sc_prop(y1, row, col)
    tx1, y2 = _tc_mid(d2, p1)
    p2 = _sc_prop(y2, row, col)
    h, y3 = _tc_layer_relu(d2, x, tx1, p2, W1, b1r, g1r, be1r)

    p3 = _sc_prop(y3, row, col)
    tx2, y4 = _tc_mid(d2, p3)
    p4 = _sc_prop(y4, row, col)
    out = _tc_layer_lin(d2, h, tx2, p4, W2, b2r, g2r, be2r)
    return out


# speedup vs baseline: 1.2079x; 1.2079x over previous
import functools

import jax
import jax.numpy as jnp
from jax import lax
from jax.experimental import pallas as pl
from jax.experimental.pallas import tpu as pltpu
from jax.experimental.pallas import tpu_sc as plsc

N = 10000
E = 320000
D = 128

NC = 2
NS = 16
NW = NC * NS
EPT = E // NW
CH = 40
NCHUNK = EPT // CH
G = 400 // CH
NG = NCHUNK // G
GE = G * CH
R = 5
PF = 3
DCH = 80
DNCHUNK = EPT // DCH
RPS = 624
ZB = 16

_mesh = plsc.VectorSubcoreMesh(core_axis_name="c", subcore_axis_name="s")


def _zero_fill(ref, nrow, ncol):
    zv = jnp.zeros((16,), jnp.float32)

    def body(i, _):
        for c in range(ncol // 16):
            ref[i, pl.ds(c * 16, 16)] = zv
        return 0

    lax.fori_loop(0, nrow, body, 0)


def _zero_acc(zbuf, acc, sid):
    z16 = zbuf.at[pl.ds(0, ZB)]
    for j in range(RPS // ZB):
        pltpu.sync_copy(z16, acc.at[pl.ds(sid * RPS + j * ZB, ZB)])

    @pl.when(sid == NS - 1)
    def _():
        pltpu.sync_copy(z16, acc.at[pl.ds(NS * RPS, 16)])


def _writeback(acc, out_hbm, cid, sid):
    pltpu.sync_copy(
        acc.at[pl.ds(sid * RPS, RPS)],
        out_hbm.at[cid, pl.ds(sid * RPS, RPS)],
    )

    @pl.when(sid == NS - 1)
    def _():
        pltpu.sync_copy(
            acc.at[pl.ds(NS * RPS, 16)],
            out_hbm.at[cid, pl.ds(NS * RPS, 16)],
        )


def _sc_prop_body(y_hbm, src_hbm, dst_hbm, out_hbm, idx_s0, idx_s1, idx_d,
                  rows, acc, sem_i, sem_g, sem_s):
    cid = lax.axis_index("c")
    sid = lax.axis_index("s")
    wid = cid * NS + sid
    idx_s = [idx_s0, idx_s1]

    def fetch_idx(group, p):
        pltpu.async_copy(src_hbm.at[pl.ds(wid * EPT + group * GE, GE)],
                         idx_s[p], sem_i.at[p])
        pltpu.async_copy(dst_hbm.at[wid, group], idx_d.at[p], sem_i.at[p])

    def wait_idx(group, p):
        pltpu.make_async_copy(src_hbm.at[pl.ds(wid * EPT + group * GE, GE)],
                              idx_s[p], sem_i.at[p]).wait()
        pltpu.make_async_copy(dst_hbm.at[wid, group], idx_d.at[p],
                              sem_i.at[p]).wait()

    def start_gather(p, l, b):
        pltpu.async_copy(y_hbm.at[idx_s[p].at[pl.ds(l * CH, CH)]], rows.at[b],
                         sem_g.at[b])

    def wait_gather(p, l, b):
        pltpu.make_async_copy(y_hbm.at[idx_s[p].at[pl.ds(l * CH, CH)]],
                              rows.at[b], sem_g.at[b]).wait()

    def start_scatter(p, l, b):
        pltpu.async_copy(rows.at[b], acc.at[idx_d.at[p, l]], sem_s.at[b],
                         add=True)

    def wait_scatter(p, l, b):
        pltpu.make_async_copy(rows.at[b], acc.at[idx_d.at[p, l]],
                              sem_s.at[b]).wait()

    def slotpos(t):
        return (t // G) % 2, t % G, t % R

    fetch_idx(0, 0)
    zbuf = rows.at[0]
    _zero_fill(zbuf, ZB, D)
    _zero_acc(zbuf, acc, sid)
    wait_idx(0, 0)

    for t in range(PF):
        start_gather(0, t, t)
    plsc.subcore_barrier()

    def step(m, t, first):
        p, l, b = slotpos(t)
        wait_gather(p, l, b)
        start_scatter(p, l, b)
        if t == 2:
            fetch_idx(2 * m + 1, 1)
        if t == G + 2:
            fetch_idx(2 * m + 2, 0)
        tj = t + PF
        if tj == G:
            wait_idx(2 * m + 1, 1)
        if tj == 2 * G:
            wait_idx(2 * m + 2, 0)
        pj, lj, bj = slotpos(tj % (2 * G))
        if not (first and tj - R < 0):
            td = tj - R
            if first:
                pd, ld, bd = slotpos(td)
                wait_scatter(pd, ld, bd)
            else:
                pd, ld, bd = slotpos(td % (2 * G))
                wait_scatter(pd, ld, bd)
        start_gather(pj, lj, bj)

    for t in range(2 * G):
        step(0, t, True)

    def sgroup(m, _):
        for t in range(2 * G):
            step(m, t, False)
        return 0

    nsg = NCHUNK // (2 * G)
    lax.fori_loop(1, nsg, sgroup, 0)

    m = nsg
    for t in range(G):
        p, l, b = slotpos(t)
        wait_gather(p, l, b)
        start_scatter(p, l, b)
        tj = t + PF
        if tj < G:
            td = tj - R
            if td >= 0:
                pd, ld, bd = slotpos(td)
                wait_scatter(pd, ld, bd)
            else:
                pd, ld, bd = slotpos((td) % (2 * G))
                wait_scatter(pd, ld, bd)
            start_gather(0, tj, tj % R)

    for t in range(G - R, G):
        p, l, b = slotpos(t)
        wait_scatter(p, l, b)

    plsc.subcore_barrier()
    _writeback(acc, out_hbm, cid, sid)


_sc_prop = pl.kernel(
    _sc_prop_body,
    out_type=jax.ShapeDtypeStruct((NC, N, D), jnp.float32),
    mesh=_mesh,
    scratch_types=[
        pltpu.VMEM((GE,), jnp.int32),
        pltpu.VMEM((GE,), jnp.int32),
        pltpu.VMEM((2, G, CH), jnp.int32),
        pltpu.VMEM((R, CH, D), jnp.float32),
        pltpu.VMEM_SHARED((N, D), jnp.float32),
        pltpu.SemaphoreType.DMA((2,)),
        pltpu.SemaphoreType.DMA((R,)),
        pltpu.SemaphoreType.DMA((R,)),
    ],
)


def _sc_deg_body(src_hbm, out_hbm, idx_s, ones, zbuf, acc, sem_i, sem_s):
    cid = lax.axis_index("c")
    sid = lax.axis_index("s")
    wid = cid * NS + sid

    ci = pltpu.async_copy(src_hbm.at[wid], idx_s, sem_i)
    _zero_fill(zbuf, ZB, D)
    ov = jnp.ones((16,), jnp.float32)

    def fill_ones(i, _):
        for c in range(D // 16):
            ones[i, pl.ds(c * 16, 16)] = ov
        return 0

    lax.fori_loop(0, DCH, fill_ones, 0)
    _zero_acc(zbuf, acc, sid)
    ci.wait()
    plsc.subcore_barrier()

    def fire(i, _):
        pltpu.async_copy(ones, acc.at[idx_s.at[i]], sem_s, add=True)
        return 0

    def drain(i, _):
        pltpu.make_async_copy(ones, acc.at[idx_s.at[i]], sem_s).wait()
        return 0

    lax.fori_loop(0, DNCHUNK, fire, 0)
    lax.fori_loop(0, DNCHUNK, drain, 0)
    plsc.subcore_barrier()
    _writeback(acc, out_hbm, cid, sid)


_sc_deg = pl.kernel(
    _sc_deg_body,
    out_type=jax.ShapeDtypeStruct((NC, N, D), jnp.float32),
    mesh=_mesh,
    scratch_types=[
        pltpu.VMEM((DNCHUNK, DCH), jnp.int32),
        pltpu.VMEM((DCH, D), jnp.float32),
        pltpu.VMEM((ZB, D), jnp.float32),
        pltpu.VMEM_SHARED((N, D), jnp.float32),
        pltpu.SemaphoreType.DMA,
        pltpu.SemaphoreType.DMA,
    ],
)


def _dis_of(d_ref):
    deg = d_ref[0][:, 0:1] + d_ref[1][:, 0:1]
    return jnp.where(deg > 0.0, lax.rsqrt(jnp.maximum(deg, 1.0)), 0.0)


def _tc_scale_body(d_ref, x_ref, o_ref):
    o_ref[...] = _dis_of(d_ref) * x_ref[...]


_tc_scale = pl.pallas_call(
    _tc_scale_body,
    out_shape=jax.ShapeDtypeStruct((N, D), jnp.float32),
)


def _tc_mid_body(d_ref, p_ref, tx_ref, y_ref):
    dis = _dis_of(d_ref)
    tx1 = (-dis) * (p_ref[0] + p_ref[1])
    tx_ref[...] = tx1
    y_ref[...] = dis * tx1


_tc_mid = pl.pallas_call(
    _tc_mid_body,
    out_shape=(
        jax.ShapeDtypeStruct((N, D), jnp.float32),
        jax.ShapeDtypeStruct((N, D), jnp.float32),
    ),
)


def _tc_layer_body(d_ref, xin_ref, tx1_ref, p_ref, w_ref, b_ref, g_ref, be_ref,
                   *refs, relu):
    dis = _dis_of(d_ref)
    sp = (-dis) * (p_ref[0] + p_ref[1])
    h = (
        jnp.dot(xin_ref[...], w_ref[0] - w_ref[2],
                preferred_element_type=jnp.float32)
        + jnp.dot(tx1_ref[...], w_ref[1], preferred_element_type=jnp.float32)
        + 2.0 * jnp.dot(sp, w_ref[2], preferred_element_type=jnp.float32)
        + b_ref[...]
    )
    m = jnp.mean(h, axis=0, keepdims=True)
    c = h - m
    v = jnp.mean(c * c, axis=0, keepdims=True)
    o = g_ref[...] * c * lax.rsqrt(v + 1e-5) + be_ref[...]
    if relu:
        o = jnp.maximum(o, 0.0)
        refs[0][...] = o
        refs[1][...] = dis * o
    else:
        refs[0][...] = o


_tc_layer_relu = pl.pallas_call(
    functools.partial(_tc_layer_body, relu=True),
    out_shape=(
        jax.ShapeDtypeStruct((N, D), jnp.float32),
        jax.ShapeDtypeStruct((N, D), jnp.float32),
    ),
)
_tc_layer_lin = pl.pallas_call(
    functools.partial(_tc_layer_body, relu=False),
    out_shape=jax.ShapeDtypeStruct((N, D), jnp.float32),
)


def _jnp_ref(x, edge_index, W1, b1, gamma1, beta1, W2, b2, gamma2, beta2,
             deg=None, prop=None):
    row, col = edge_index[0], edge_index[1]
    n = x.shape[0]
    if deg is None:
        deg = jnp.zeros((n,), jnp.float32).at[row].add(1.0)
    dis = jnp.where(deg > 0, lax.rsqrt(jnp.maximum(deg, 1e-12)), 0.0)
    if prop is None:
        def prop(y):
            return jnp.zeros((n, y.shape[1]), jnp.float32).at[col].add(
                jnp.take(y, row, axis=0))

    def S(y):
        return -dis[:, None] * prop(dis[:, None] * y)

    def layer(xin, W, b):
        tx1 = S(xin)
        sp = S(tx1)
        return xin @ (W[0] - W[2]) + tx1 @ W[1] + 2.0 * (sp @ W[2]) + b

    def bn(h, g, b):
        m = jnp.mean(h, axis=0)
        v = jnp.mean((h - m) ** 2, axis=0)
        return g * (h - m) / jnp.sqrt(v + 1e-5) + b

    h = jax.nn.relu(bn(layer(x, W1, b1), gamma1, beta1))
    return bn(layer(h, W2, b2), gamma2, beta2)


def kernel(x, edge_index, W1, b1, gamma1, beta1, W2, b2, gamma2, beta2):
    row3 = edge_index[0].reshape(NW, DNCHUNK, DCH)
    row = edge_index[0]
    col = edge_index[1].reshape(NW, NG, G, CH)
    b1r = b1.reshape(1, D)
    g1r = gamma1.reshape(1, D)
    be1r = beta1.reshape(1, D)
    b2r = b2.reshape(1, D)
    g2r = gamma2.reshape(1, D)
    be2r = beta2.reshape(1, D)

    d2 = _sc_deg(row3)

    y1 = _tc_scale(d2, x)
    p1 = _sc_prop(y1, row, col)
    tx1, y2 = _tc_mid(d2, p1)
    p2 = _sc_prop(y2, row, col)
    h, y3 = _tc_layer_relu(d2, x, tx1, p2, W1, b1r, g1r, be1r)

    p3 = _sc_prop(y3, row, col)
    tx2, y4 = _tc_mid(d2, p3)
    p4 = _sc_prop(y4, row, col)
    out = _tc_layer_lin(d2, h, tx2, p4, W2, b2r, g2r, be2r)
    return out


# speedup vs baseline: 1.2086x; 1.0006x over previous
import functools

import jax
import jax.numpy as jnp
from jax import lax
from jax.experimental import pallas as pl
from jax.experimental.pallas import tpu as pltpu
from jax.experimental.pallas import tpu_sc as plsc

N = 10000
E = 320000
D = 128

NC = 2
NS = 16
NW = NC * NS
EPT = E // NW
CH = 40
NCHUNK = EPT // CH
G = 400 // CH
NG = NCHUNK // G
GE = G * CH
R = 5
PF = 3
DCH = 80
DNCHUNK = EPT // DCH
RPS = 624
ZB = 16

_mesh = plsc.VectorSubcoreMesh(core_axis_name="c", subcore_axis_name="s")


def _zero_fill(ref, nrow, ncol):
    zv = jnp.zeros((16,), jnp.float32)

    def body(i, _):
        for c in range(ncol // 16):
            ref[i, pl.ds(c * 16, 16)] = zv
        return 0

    lax.fori_loop(0, nrow, body, 0)


def _zero_acc(zbuf, acc, sid):
    z16 = zbuf.at[pl.ds(0, ZB)]
    for j in range(RPS // ZB):
        pltpu.sync_copy(z16, acc.at[pl.ds(sid * RPS + j * ZB, ZB)])

    @pl.when(sid == NS - 1)
    def _():
        pltpu.sync_copy(z16, acc.at[pl.ds(NS * RPS, 16)])


def _writeback(acc, out_hbm, cid, sid):
    pltpu.sync_copy(
        acc.at[pl.ds(sid * RPS, RPS)],
        out_hbm.at[cid, pl.ds(sid * RPS, RPS)],
    )

    @pl.when(sid == NS - 1)
    def _():
        pltpu.sync_copy(
            acc.at[pl.ds(NS * RPS, 16)],
            out_hbm.at[cid, pl.ds(NS * RPS, 16)],
        )


def _sc_prop_body(y_hbm, src_hbm, dst_hbm, out_hbm, idx_s0, idx_s1, idx_d,
                  rows, acc, sem_i, sem_g, sem_s):
    cid = lax.axis_index("c")
    sid = lax.axis_index("s")
    wid = cid * NS + sid
    idx_s = [idx_s0, idx_s1]

    def fetch_idx(group, p):
        pltpu.async_copy(src_hbm.at[pl.ds(wid * EPT + group * GE, GE)],
                         idx_s[p], sem_i.at[p])
        pltpu.async_copy(dst_hbm.at[wid, group], idx_d.at[p], sem_i.at[p])

    def wait_idx(group, p):
        pltpu.make_async_copy(src_hbm.at[pl.ds(wid * EPT + group * GE, GE)],
                              idx_s[p], sem_i.at[p]).wait()
        pltpu.make_async_copy(dst_hbm.at[wid, group], idx_d.at[p],
                              sem_i.at[p]).wait()

    def start_gather(p, l, b):
        pltpu.async_copy(y_hbm.at[idx_s[p].at[pl.ds(l * CH, CH)]], rows.at[b],
                         sem_g.at[b])

    def wait_gather(p, l, b):
        pltpu.make_async_copy(y_hbm.at[idx_s[p].at[pl.ds(l * CH, CH)]],
                              rows.at[b], sem_g.at[b]).wait()

    def start_scatter(p, l, b):
        pltpu.async_copy(rows.at[b], acc.at[idx_d.at[p, l]], sem_s.at[b],
                         add=True)

    def wait_scatter(p, l, b):
        pltpu.make_async_copy(rows.at[b], acc.at[idx_d.at[p, l]],
                              sem_s.at[b]).wait()

    def slotpos(t):
        return (t // G) % 2, t % G, t % R

    fetch_idx(0, 0)
    zbuf = rows.at[0]
    _zero_fill(zbuf, ZB, D)
    _zero_acc(zbuf, acc, sid)
    wait_idx(0, 0)

    for t in range(PF):
        start_gather(0, t, t)
    plsc.subcore_barrier()

    def step(m, t, first):
        p, l, b = slotpos(t)
        wait_gather(p, l, b)
        start_scatter(p, l, b)
        if t == 2:
            fetch_idx(2 * m + 1, 1)
        if t == G + 2:
            fetch_idx(2 * m + 2, 0)
        tj = t + PF
        if tj == G:
            wait_idx(2 * m + 1, 1)
        if tj == 2 * G:
            wait_idx(2 * m + 2, 0)
        pj, lj, bj = slotpos(tj % (2 * G))
        if not (first and tj - R < 0):
            td = tj - R
            if first:
                pd, ld, bd = slotpos(td)
                wait_scatter(pd, ld, bd)
            else:
                pd, ld, bd = slotpos(td % (2 * G))
                wait_scatter(pd, ld, bd)
        start_gather(pj, lj, bj)

    for t in range(2 * G):
        step(0, t, True)

    def sgroup(m, _):
        for t in range(2 * G):
            step(m, t, False)
        return 0

    nsg = NCHUNK // (2 * G)
    lax.fori_loop(1, nsg, sgroup, 0)

    m = nsg
    for t in range(G):
        p, l, b = slotpos(t)
        wait_gather(p, l, b)
        start_scatter(p, l, b)
        tj = t + PF
        if tj < G:
            td = tj - R
            if td >= 0:
                pd, ld, bd = slotpos(td)
                wait_scatter(pd, ld, bd)
            else:
                pd, ld, bd = slotpos((td) % (2 * G))
                wait_scatter(pd, ld, bd)
            start_gather(0, tj, tj % R)

    for t in range(G - R, G):
        p, l, b = slotpos(t)
        wait_scatter(p, l, b)

    plsc.subcore_barrier()
    _writeback(acc, out_hbm, cid, sid)


_sc_prop = pl.kernel(
    _sc_prop_body,
    out_type=jax.ShapeDtypeStruct((NC, N, D), jnp.float32),
    mesh=_mesh,
    scratch_types=[
        pltpu.VMEM((GE,), jnp.int32),
        pltpu.VMEM((GE,), jnp.int32),
        pltpu.VMEM((2, G, CH), jnp.int32),
        pltpu.VMEM((R, CH, D), jnp.float32),
        pltpu.VMEM_SHARED((N, D), jnp.float32),
        pltpu.SemaphoreType.DMA((2,)),
        pltpu.SemaphoreType.DMA((R,)),
        pltpu.SemaphoreType.DMA((R,)),
    ],
)


def _sc_deg_body(src_hbm, out_hbm, idx_s, ones, zbuf, acc, sem_i, sem_s):
    cid = lax.axis_index("c")
    sid = lax.axis_index("s")
    wid = cid * NS + sid

    ci = pltpu.async_copy(src_hbm.at[wid], idx_s, sem_i)
    _zero_fill(zbuf, ZB, D)
    ov = jnp.ones((16,), jnp.float32)

    def fill_ones(i, _):
        for c in range(D // 16):
            ones[i, pl.ds(c * 16, 16)] = ov
        return 0

    lax.fori_loop(0, DCH, fill_ones, 0)
    _zero_acc(zbuf, acc, sid)
    ci.wait()
    plsc.subcore_barrier()

    def fire(i, _):
        pltpu.async_copy(ones, acc.at[idx_s.at[i]], sem_s, add=True)
        return 0

    def drain(i, _):
        pltpu.make_async_copy(ones, acc.at[idx_s.at[i]], sem_s).wait()
        return 0

    lax.fori_loop(0, DNCHUNK, fire, 0)
    lax.fori_loop(0, DNCHUNK, drain, 0)
    plsc.subcore_barrier()
    _writeback(acc, out_hbm, cid, sid)


_sc_deg = pl.kernel(
    _sc_deg_body,
    out_type=jax.ShapeDtypeStruct((NC, N, D), jnp.float32),
    mesh=_mesh,
    scratch_types=[
        pltpu.VMEM((DNCHUNK, DCH), jnp.int32),
        pltpu.VMEM((DCH, D), jnp.float32),
        pltpu.VMEM((ZB, D), jnp.float32),
        pltpu.VMEM_SHARED((N, D), jnp.float32),
        pltpu.SemaphoreType.DMA,
        pltpu.SemaphoreType.DMA,
    ],
)


def _dis_of(d_ref):
    deg = d_ref[0][:, 0:1] + d_ref[1][:, 0:1]
    return jnp.where(deg > 0.0, lax.rsqrt(jnp.maximum(deg, 1.0)), 0.0)


def _tc_scale_body(d_ref, x_ref, o_ref):
    o_ref[...] = _dis_of(d_ref) * x_ref[...]


_tc_scale = pl.pallas_call(
    _tc_scale_body,
    out_shape=jax.ShapeDtypeStruct((N, D), jnp.float32),
)


def _tc_mid_body(d_ref, p_ref, tx_ref, y_ref):
    dis = _dis_of(d_ref)
    tx1 = (-dis) * (p_ref[0] + p_ref[1])
    tx_ref[...] = tx1
    y_ref[...] = dis * tx1


_tc_mid = pl.pallas_call(
    _tc_mid_body,
    out_shape=(
        jax.ShapeDtypeStruct((N, D), jnp.float32),
        jax.ShapeDtypeStruct((N, D), jnp.float32),
    ),
)


def _tc_layer_body(d_ref, xin_ref, tx1_ref, p_ref, w_ref, b_ref, g_ref, be_ref,
                   *refs, relu):
    dis = _dis_of(d_ref)
    sp = (-dis) * (p_ref[0] + p_ref[1])
    h = (
        jnp.dot(xin_ref[...], w_ref[0] - w_ref[2],
                preferred_element_type=jnp.float32)
        + jnp.dot(tx1_ref[...], w_ref[1], preferred_element_type=jnp.float32)
        + 2.0 * jnp.dot(sp, w_ref[2], preferred_element_type=jnp.float32)
        + b_ref[...]
    )
    m = jnp.mean(h, axis=0, keepdims=True)
    c = h - m
    v = jnp.mean(c * c, axis=0, keepdims=True)
    o = g_ref[...] * c * lax.rsqrt(v + 1e-5) + be_ref[...]
    if relu:
        o = jnp.maximum(o, 0.0)
        refs[0][...] = o
        refs[1][...] = dis * o
    else:
        refs[0][...] = o


_tc_layer_relu = pl.pallas_call(
    functools.partial(_tc_layer_body, relu=True),
    out_shape=(
        jax.ShapeDtypeStruct((N, D), jnp.float32),
        jax.ShapeDtypeStruct((N, D), jnp.float32),
    ),
)
_tc_layer_lin = pl.pallas_call(
    functools.partial(_tc_layer_body, relu=False),
    out_shape=jax.ShapeDtypeStruct((N, D), jnp.float32),
)


def kernel(x, edge_index, W1, b1, gamma1, beta1, W2, b2, gamma2, beta2):
    row3 = edge_index[0].reshape(NW, DNCHUNK, DCH)
    row = edge_index[0]
    col = edge_index[1].reshape(NW, NG, G, CH)
    b1r = b1.reshape(1, D)
    g1r = gamma1.reshape(1, D)
    be1r = beta1.reshape(1, D)
    b2r = b2.reshape(1, D)
    g2r = gamma2.reshape(1, D)
    be2r = beta2.reshape(1, D)

    d2 = _sc_deg(row3)

    y1 = _tc_scale(d2, x)
    p1 = _sc_prop(y1, row, col)
    tx1, y2 = _tc_mid(d2, p1)
    p2 = _sc_prop(y2, row, col)
    h, y3 = _tc_layer_relu(d2, x, tx1, p2, W1, b1r, g1r, be1r)

    p3 = _sc_prop(y3, row, col)
    tx2, y4 = _tc_mid(d2, p3)
    p4 = _sc_prop(y4, row, col)
    out = _tc_layer_lin(d2, h, tx2, p4, W2, b2r, g2r, be2r)
    return out
